# baseline (speedup 1.0000x reference)
import functools

import jax
import jax.numpy as jnp
from jax import lax
from jax.experimental import pallas as pl
from jax.experimental.pallas import tpu as pltpu
from jax.experimental.pallas import tpu_sc as plsc

_POOL = 4096
_K = 4
_L = 8
_D = 768
_B = 1024
_BB = 1024
_NW = 32
_CH = 8
_NBUF = 2
_HALVES = 1


def _make_topk(nb):
    grid = nb // _BB

    def body(q_ref, k_ref, sim_ref, idx_ref, acc_ref):
        qn = q_ref[...]
        kn = k_ref[...]
        sim = jax.lax.dot_general(
            qn, kn, (((1,), (1,)), ((), ())),
            preferred_element_type=jnp.float32)

        sim_ref[...] = jnp.zeros((_BB, 128), jnp.float32)
        idx_ref[...] = jnp.zeros((_BB, 128), jnp.int32)
        col_ids = jax.lax.broadcasted_iota(jnp.int32, (_BB, _POOL), 1)
        work = sim
        total = jnp.float32(0.0)
        big = jnp.int32(2**30)
        for t in range(_K):
            m = jnp.max(work, axis=1, keepdims=True)
            cand = jnp.where(work == m, col_ids, big)
            sel = jnp.min(cand, axis=1, keepdims=True)
            sim_ref[:, t:t + 1] = m
            idx_ref[:, t:t + 1] = sel
            total = total + jnp.sum(m)
            work = jnp.where(col_ids == sel, -jnp.inf, work)

        @pl.when(pl.program_id(0) == 0)
        def _():
            acc_ref[0, 0] = 0.0

        acc_ref[0, 0] += total

    return pl.pallas_call(
        body,
        grid=(grid,),
        in_specs=[
            pl.BlockSpec((_BB, _D), lambda i: (i, 0)),
            pl.BlockSpec((_POOL, _D), lambda i: (0, 0)),
        ],
        out_specs=[
            pl.BlockSpec((_BB, 128), lambda i: (i, 0)),
            pl.BlockSpec((_BB, 128), lambda i: (i, 0)),
            pl.BlockSpec(memory_space=pltpu.SMEM, block_shape=(1, 1),
                         index_map=lambda i: (0, 0)),
        ],
        out_shape=[
            jax.ShapeDtypeStruct((nb, 128), jnp.float32),
            jax.ShapeDtypeStruct((nb, 128), jnp.int32),
            jax.ShapeDtypeStruct((1, 1), jnp.float32),
        ],
    )


def _make_sc_gather(n_rows):
    bpw = n_rows // _NW
    nchunk = bpw // _CH
    mesh = plsc.VectorSubcoreMesh(core_axis_name="c", subcore_axis_name="s")

    def body(table_hbm, idx_hbm, out_hbm, idx_v, bufs, *sems):
        wid = lax.axis_index("s") * 2 + lax.axis_index("c")
        base = wid * bpw
        pltpu.sync_copy(idx_hbm.at[wid], idx_v)
        gsems = sems[:_NBUF]
        wsems = sems[_NBUF:]

        def gather(c):
            p = c % _NBUF
            return pltpu.async_copy(
                table_hbm.at[idx_v.at[c]], bufs.at[p],
                gsems[p])

        writes = [None] * _NBUF
        gathers = [None] * _NBUF
        gathers[0] = gather(0)
        for c in range(nchunk):
            p = c % _NBUF
            gathers[p].wait()
            if c + 1 < nchunk:
                pn = (c + 1) % _NBUF
                if writes[pn] is not None:
                    writes[pn].wait()
                gathers[pn] = gather(c + 1)
            writes[p] = pltpu.async_copy(
                bufs.at[p], out_hbm.at[pl.ds(base + c * _CH, _CH)], wsems[p])
        for w in writes:
            if w is not None:
                w.wait()

    f = functools.partial(
        pl.kernel,
        mesh=mesh,
        out_type=jax.ShapeDtypeStruct((n_rows, _L, _D), jnp.float32),
        scratch_types=(
            [pltpu.VMEM((nchunk, _CH), jnp.int32),
             pltpu.VMEM((_NBUF, _CH, _L, _D), jnp.float32)]
            + [pltpu.SemaphoreType.DMA] * (2 * _NBUF)),
    )(body)
    return lambda values, flat_idx: f(
        values, flat_idx.reshape(_NW, nchunk, _CH))


def _unit_rows(x):
    n = jnp.linalg.norm(x, axis=-1, keepdims=True)
    return x / jnp.maximum(n, 1e-12)


@jax.jit
def kernel(query, keys, values):
    qn = _unit_rows(query)
    kn = _unit_rows(keys)
    nb = _B // _HALVES
    topk = _make_topk(nb)
    gather = _make_sc_gather(nb * _K)
    parts = []
    accs = []
    for h in range(_HALVES):
        top_sim_p, top_idx_p, acc = topk(qn[h * nb:(h + 1) * nb], kn)
        flat_idx = top_idx_p[:, :_K].reshape(nb * _K)
        parts.append(gather(values, flat_idx))
        accs.append(acc[0, 0])
    if _HALVES == 1:
        selected = parts[0]
    else:
        selected = jnp.concatenate(parts, axis=0)
    reduce_sim = sum(accs) / jnp.float32(_B * _K)
    return selected.reshape(_B, _K * _L, _D), reduce_sim


# speedup vs baseline: 1.0056x; 1.0056x over previous
import functools

import jax
import jax.numpy as jnp
from jax import lax
from jax.experimental import pallas as pl
from jax.experimental.pallas import tpu as pltpu
from jax.experimental.pallas import tpu_sc as plsc

_POOL = 4096
_K = 4
_L = 8
_D = 768
_B = 1024
_BB = 512
_NW = 32
_CH = 8
_NBUF = 2


def _make_topk(nb):
    grid = nb // _BB

    def body(q_ref, k_ref, sim_ref, idx_ref, acc_ref):
        qn = q_ref[...]
        kn = k_ref[...]
        sim = jax.lax.dot_general(
            qn, kn, (((1,), (1,)), ((), ())),
            preferred_element_type=jnp.float32)

        sim_ref[...] = jnp.zeros((_BB, 128), jnp.float32)
        idx_ref[...] = jnp.zeros((_BB, 128), jnp.int32)
        col_ids = jax.lax.broadcasted_iota(jnp.int32, (_BB, _POOL), 1)
        work = sim
        total = jnp.float32(0.0)
        big = jnp.int32(2**30)
        for t in range(_K):
            m = jnp.max(work, axis=1, keepdims=True)
            cand = jnp.where(work == m, col_ids, big)
            sel = jnp.min(cand, axis=1, keepdims=True)
            sim_ref[:, t:t + 1] = m
            idx_ref[:, t:t + 1] = sel
            total = total + jnp.sum(m)
            work = jnp.where(col_ids == sel, -jnp.inf, work)

        @pl.when(pl.program_id(0) == 0)
        def _():
            acc_ref[0, 0] = 0.0

        acc_ref[0, 0] += total

    return pl.pallas_call(
        body,
        grid=(grid,),
        in_specs=[
            pl.BlockSpec((_BB, _D), lambda i: (i, 0)),
            pl.BlockSpec((_POOL, _D), lambda i: (0, 0)),
        ],
        out_specs=[
            pl.BlockSpec((_BB, 128), lambda i: (i, 0)),
            pl.BlockSpec((_BB, 128), lambda i: (i, 0)),
            pl.BlockSpec(memory_space=pltpu.SMEM, block_shape=(1, 1),
                         index_map=lambda i: (0, 0)),
        ],
        out_shape=[
            jax.ShapeDtypeStruct((nb, 128), jnp.float32),
            jax.ShapeDtypeStruct((nb, 128), jnp.int32),
            jax.ShapeDtypeStruct((1, 1), jnp.float32),
        ],
    )


def _make_sc_gather(n_rows):
    bpw = n_rows // _NW
    nchunk = bpw // _CH
    mesh = plsc.VectorSubcoreMesh(core_axis_name="c", subcore_axis_name="s")

    def body(table_hbm, idx_hbm, out_hbm, idx_v, bufs, *sems):
        wid = lax.axis_index("s") * 2 + lax.axis_index("c")
        base = wid * bpw
        pltpu.sync_copy(idx_hbm.at[wid], idx_v)
        gsems = sems[:_NBUF]
        wsems = sems[_NBUF:]

        def gather(c):
            p = c % _NBUF
            return pltpu.async_copy(
                table_hbm.at[idx_v.at[c]], bufs.at[p],
                gsems[p])

        writes = [None] * _NBUF
        gathers = [None] * _NBUF
        gathers[0] = gather(0)
        for c in range(nchunk):
            p = c % _NBUF
            gathers[p].wait()
            if c + 1 < nchunk:
                pn = (c + 1) % _NBUF
                if writes[pn] is not None:
                    writes[pn].wait()
                gathers[pn] = gather(c + 1)
            writes[p] = pltpu.async_copy(
                bufs.at[p], out_hbm.at[pl.ds(base + c * _CH, _CH)], wsems[p])
        for w in writes:
            if w is not None:
                w.wait()

    f = functools.partial(
        pl.kernel,
        mesh=mesh,
        out_type=jax.ShapeDtypeStruct((n_rows, _L, _D), jnp.float32),
        scratch_types=(
            [pltpu.VMEM((nchunk, _CH), jnp.int32),
             pltpu.VMEM((_NBUF, _CH, _L, _D), jnp.float32)]
            + [pltpu.SemaphoreType.DMA] * (2 * _NBUF)),
    )(body)
    return lambda values, flat_idx: f(
        values, flat_idx.reshape(_NW, nchunk, _CH))


def _unit_rows(x):
    n = jnp.linalg.norm(x, axis=-1, keepdims=True)
    return x / jnp.maximum(n, 1e-12)


@jax.jit
def kernel(query, keys, values):
    qn = _unit_rows(query)
    kn = _unit_rows(keys)
    top_sim_p, top_idx_p, acc = _make_topk(_B)(qn, kn)
    flat_idx = top_idx_p[:, :_K].reshape(_B * _K)
    selected = _make_sc_gather(_B * _K)(values, flat_idx)
    reduce_sim = acc[0, 0] / jnp.float32(_B * _K)
    return selected.reshape(_B, _K * _L, _D), reduce_sim
